# baseline (speedup 1.0000x reference)
import numpy as np
import jax
import jax.numpy as jnp
from jax.experimental import pallas as pl
from jax.experimental.pallas import tpu as pltpu

N_NODES = 4096
IN_F = 128
OUT_F = 128
_RAW_WORDS = 256
_Q = N_NODES // 4

_rng = np.random.RandomState(0)
_U0 = int(_rng.randint(0, N_NODES))
_RAW = (
    _rng.randint(0, 2 ** 32, size=_RAW_WORDS, dtype=np.uint32)
    .view(np.int32)
    .reshape(1, _RAW_WORDS)
)


def _sample_idx(raw, ptr, deg):
    rmax = jnp.maximum(deg, 1) - 1
    mask = rmax
    for s in (1, 2, 4, 8, 16):
        mask = mask | (mask >> s)
    masked = raw & mask
    pos = jax.lax.broadcasted_iota(jnp.int32, (1, _RAW_WORDS), 1)
    accept = (masked <= rmax) & (pos >= ptr)
    key = (pos * 16384 + masked).astype(jnp.float32)
    kmin = jnp.min(jnp.where(accept, key, jnp.float32(2 ** 23)))
    ki = kmin.astype(jnp.int32)
    p = ki >> 14
    idx = ki & 16383
    idx = jnp.where(rmax == 0, jnp.int32(0), idx)
    new_ptr = jnp.where(rmax == 0, ptr, p + 1)
    return idx, new_ptr


def _make_tris():
    t_tri = (
        jax.lax.broadcasted_iota(jnp.int32, (128, 128), 0)
        <= jax.lax.broadcasted_iota(jnp.int32, (128, 128), 1)
    ).astype(jnp.float32)
    s_tri = (
        jax.lax.broadcasted_iota(jnp.int32, (32, 32), 1)
        < jax.lax.broadcasted_iota(jnp.int32, (32, 32), 0)
    ).astype(jnp.float32)
    flat = (
        jax.lax.broadcasted_iota(jnp.int32, (32, 128), 0) * 128
        + jax.lax.broadcasted_iota(jnp.int32, (32, 128), 1)
    ).astype(jnp.float32)
    return t_tri, s_tri, flat


def _select_kth(m2, idx, tris):
    t_tri, s_tri, flat = tris
    prefix = jnp.dot(m2, t_tri, preferred_element_type=jnp.float32)
    rows_before = jnp.dot(s_tri, prefix, preferred_element_type=jnp.float32)
    cum = prefix + rows_before[:, 127:128]
    tgt = (idx + 1).astype(jnp.float32)
    hit = m2 * (jnp.abs(cum - tgt) < 0.5).astype(jnp.float32)
    return jnp.sum(hit * flat).astype(jnp.int32)


def _body(w_ref, b_ref, raw_ref, adj_ref, x_any, out_any,
          xbuf, obuf, row1_scr, row2_scr, x0_scr, xa_scr, xb_scr, y0_scr,
          sem_adj, sem_xq, sem_oq, sem_row):
    w1 = w_ref[0:IN_F, :]
    bias = b_ref[0:1, :]

    def xq_copy(q):
        return pltpu.make_async_copy(
            x_any.at[pl.ds(q * _Q, _Q), :], xbuf.at[pl.ds(q * _Q, _Q), :],
            sem_xq.at[q])

    def oq_copy(q):
        return pltpu.make_async_copy(
            obuf.at[pl.ds(q * _Q, _Q), :], out_any.at[pl.ds(q * _Q, _Q), :],
            sem_oq.at[q])

    def mm_quarter(q):
        main = (
            jnp.dot(xbuf[pl.ds(q * _Q, _Q), :], w1,
                    preferred_element_type=jnp.float32)
            + bias
        )
        obuf[pl.ds(q * _Q, _Q), :] = jnp.maximum(main, 0.0)

    cp1 = pltpu.make_async_copy(
        adj_ref.at[pl.ds(_U0, 1), :], row1_scr, sem_adj)
    cp1.start()
    pltpu.make_async_copy(
        x_any.at[pl.ds(0, 1), :], x0_scr, sem_row.at[0]).start()
    xq_copy(0).start()

    tris = _make_tris()

    cp1.wait()
    m1 = row1_scr[...].reshape(32, 128)
    deg1 = jnp.sum(m1).astype(jnp.int32)
    idx1, ptr1 = _sample_idx(raw_ref[...], jnp.int32(0), deg1)
    v1 = _select_kth(m1, idx1, tris)
    has1 = deg1 > 0
    ptr = jnp.where(has1, ptr1, jnp.int32(0))
    u2 = jnp.where(has1, v1, jnp.int32(_U0))
    cp2 = pltpu.make_async_copy(
        adj_ref.at[pl.ds(u2, 1), :], row2_scr, sem_adj)
    cp2.start()
    pltpu.make_async_copy(
        x_any.at[pl.ds(v1, 1), :], xa_scr, sem_row.at[1]).start()
    xq_copy(1).start()

    xq_copy(0).wait()
    mm_quarter(0)
    oq_copy(0).start()
    xq_copy(2).start()

    cp2.wait()
    m2 = row2_scr[...].reshape(32, 128)
    deg2 = jnp.sum(m2).astype(jnp.int32)
    idx2, _ = _sample_idx(raw_ref[...], ptr, deg2)
    v2 = _select_kth(m2, idx2, tris)
    has2 = has1 & (deg2 > 0)
    pltpu.make_async_copy(
        x_any.at[pl.ds(v2, 1), :], xb_scr, sem_row.at[2]).start()

    xq_copy(1).wait()
    mm_quarter(1)
    oq_copy(1).start()
    xq_copy(3).start()

    pltpu.make_async_copy(
        x_any.at[pl.ds(0, 1), :], x0_scr, sem_row.at[0]).wait()
    pltpu.make_async_copy(
        x_any.at[pl.ds(v1, 1), :], xa_scr, sem_row.at[1]).wait()
    pltpu.make_async_copy(
        x_any.at[pl.ds(v2, 1), :], xb_scr, sem_row.at[2]).wait()
    f1 = has1.astype(jnp.float32)
    f2 = has2.astype(jnp.float32)
    cnt = f1 + f2
    acc = f1 * xa_scr[...] + f2 * xb_scr[...]
    row0 = jnp.where(cnt > 0, acc / jnp.maximum(cnt, 1.0), x0_scr[...])
    w2 = w_ref[IN_F:, :]
    y0 = (
        jnp.dot(xbuf[0:1, :], w1, preferred_element_type=jnp.float32)
        + jnp.dot(row0, w2, preferred_element_type=jnp.float32)
        + bias
    )
    y0_scr[...] = jnp.maximum(y0, 0.0)

    xq_copy(2).wait()
    mm_quarter(2)
    oq_copy(2).start()
    xq_copy(3).wait()
    mm_quarter(3)
    oq_copy(3).start()

    oq_copy(0).wait()
    fix = pltpu.make_async_copy(y0_scr, out_any.at[pl.ds(0, 1), :], sem_adj)
    fix.start()
    fix.wait()
    oq_copy(1).wait()
    oq_copy(2).wait()
    oq_copy(3).wait()


def kernel(x, adj, weight, bias):
    bias2 = bias.reshape(1, OUT_F)
    return pl.pallas_call(
        _body,
        out_shape=jax.ShapeDtypeStruct((N_NODES, OUT_F), jnp.float32),
        in_specs=[
            pl.BlockSpec(memory_space=pltpu.VMEM),
            pl.BlockSpec(memory_space=pltpu.VMEM),
            pl.BlockSpec(memory_space=pltpu.VMEM),
            pl.BlockSpec(memory_space=pl.ANY),
            pl.BlockSpec(memory_space=pl.ANY),
        ],
        out_specs=pl.BlockSpec(memory_space=pl.ANY),
        scratch_shapes=[
            pltpu.VMEM((N_NODES, IN_F), jnp.float32),
            pltpu.VMEM((N_NODES, OUT_F), jnp.float32),
            pltpu.VMEM((1, N_NODES), jnp.float32),
            pltpu.VMEM((1, N_NODES), jnp.float32),
            pltpu.VMEM((1, IN_F), jnp.float32),
            pltpu.VMEM((1, IN_F), jnp.float32),
            pltpu.VMEM((1, IN_F), jnp.float32),
            pltpu.VMEM((1, OUT_F), jnp.float32),
            pltpu.SemaphoreType.DMA,
            pltpu.SemaphoreType.DMA((4,)),
            pltpu.SemaphoreType.DMA((4,)),
            pltpu.SemaphoreType.DMA((3,)),
        ],
    )(weight, bias2, jnp.asarray(_RAW), adj, x)


# speedup vs baseline: 1.1114x; 1.1114x over previous
import numpy as np
import jax
import jax.numpy as jnp
from jax.experimental import pallas as pl
from jax.experimental.pallas import tpu as pltpu

N_NODES = 4096
IN_F = 128
OUT_F = 128
_RAW_WORDS = 256
_Q = N_NODES // 4

_rng = np.random.RandomState(0)
_U0 = int(_rng.randint(0, N_NODES))
_RAW = (
    _rng.randint(0, 2 ** 32, size=_RAW_WORDS, dtype=np.uint32)
    .view(np.int32)
    .reshape(1, _RAW_WORDS)
)


def _sample_idx(raw, ptr, deg):
    rmax = jnp.maximum(deg, 1) - 1
    mask = rmax
    for s in (1, 2, 4, 8, 16):
        mask = mask | (mask >> s)
    masked = raw & mask
    pos = jax.lax.broadcasted_iota(jnp.int32, (1, _RAW_WORDS), 1)
    accept = (masked <= rmax) & (pos >= ptr)
    key = (pos * 16384 + masked).astype(jnp.float32)
    kmin = jnp.min(jnp.where(accept, key, jnp.float32(2 ** 23)))
    ki = kmin.astype(jnp.int32)
    p = ki >> 14
    idx = ki & 16383
    idx = jnp.where(rmax == 0, jnp.int32(0), idx)
    new_ptr = jnp.where(rmax == 0, ptr, p + 1)
    return idx, new_ptr


def _make_tris():
    t_tri = (
        jax.lax.broadcasted_iota(jnp.int32, (128, 128), 0)
        <= jax.lax.broadcasted_iota(jnp.int32, (128, 128), 1)
    ).astype(jnp.float32)
    s_tri = (
        jax.lax.broadcasted_iota(jnp.int32, (32, 32), 1)
        < jax.lax.broadcasted_iota(jnp.int32, (32, 32), 0)
    ).astype(jnp.float32)
    flat = (
        jax.lax.broadcasted_iota(jnp.int32, (32, 128), 0) * 128
        + jax.lax.broadcasted_iota(jnp.int32, (32, 128), 1)
    ).astype(jnp.float32)
    return t_tri, s_tri, flat


def _select_kth(m2, idx, tris):
    t_tri, s_tri, flat = tris
    prefix = jnp.dot(m2, t_tri, preferred_element_type=jnp.float32)
    rows_before = jnp.dot(s_tri, prefix, preferred_element_type=jnp.float32)
    cum = prefix + rows_before[:, 127:128]
    tgt = (idx + 1).astype(jnp.float32)
    hit = m2 * (jnp.abs(cum - tgt) < 0.5).astype(jnp.float32)
    return jnp.sum(hit * flat).astype(jnp.int32)


def _body(w_ref, b_ref, raw_ref, adj_ref, x_any, out_any,
          xbuf, obuf, row1_scr, row2_scr, x0_scr, xa_scr, xb_scr,
          sem_adj, sem_xq, sem_oq, sem_row):
    w1 = w_ref[0:IN_F, :]
    bias = b_ref[0:1, :]

    def xq_copy(q):
        return pltpu.make_async_copy(
            x_any.at[pl.ds(q * _Q, _Q), :], xbuf.at[pl.ds(q * _Q, _Q), :],
            sem_xq.at[q])

    def oq_copy(q):
        return pltpu.make_async_copy(
            obuf.at[pl.ds(q * _Q, _Q), :], out_any.at[pl.ds(q * _Q, _Q), :],
            sem_oq.at[q])

    def mm_quarter(q):
        main = (
            jnp.dot(xbuf[pl.ds(q * _Q, _Q), :], w1,
                    preferred_element_type=jnp.float32)
            + bias
        )
        obuf[pl.ds(q * _Q, _Q), :] = jnp.maximum(main, 0.0)

    cp1 = pltpu.make_async_copy(
        adj_ref.at[pl.ds(_U0, 1), :], row1_scr, sem_adj)
    cp1.start()
    pltpu.make_async_copy(
        x_any.at[pl.ds(0, 1), :], x0_scr, sem_row.at[0]).start()
    xq_copy(3).start()
    xq_copy(2).start()

    tris = _make_tris()

    cp1.wait()
    m1 = row1_scr[...].reshape(32, 128)
    deg1 = jnp.sum(m1).astype(jnp.int32)
    idx1, ptr1 = _sample_idx(raw_ref[...], jnp.int32(0), deg1)
    v1 = _select_kth(m1, idx1, tris)
    has1 = deg1 > 0
    ptr = jnp.where(has1, ptr1, jnp.int32(0))
    u2 = jnp.where(has1, v1, jnp.int32(_U0))
    cp2 = pltpu.make_async_copy(
        adj_ref.at[pl.ds(u2, 1), :], row2_scr, sem_adj)
    cp2.start()
    pltpu.make_async_copy(
        x_any.at[pl.ds(v1, 1), :], xa_scr, sem_row.at[1]).start()
    xq_copy(1).start()

    xq_copy(3).wait()
    mm_quarter(3)
    oq_copy(3).start()
    xq_copy(0).start()
    xq_copy(2).wait()
    mm_quarter(2)
    oq_copy(2).start()

    cp2.wait()
    m2 = row2_scr[...].reshape(32, 128)
    deg2 = jnp.sum(m2).astype(jnp.int32)
    idx2, _ = _sample_idx(raw_ref[...], ptr, deg2)
    v2 = _select_kth(m2, idx2, tris)
    has2 = has1 & (deg2 > 0)
    pltpu.make_async_copy(
        x_any.at[pl.ds(v2, 1), :], xb_scr, sem_row.at[2]).start()
    xq_copy(1).wait()
    mm_quarter(1)
    oq_copy(1).start()

    xq_copy(0).wait()
    mm_quarter(0)
    pltpu.make_async_copy(
        x_any.at[pl.ds(0, 1), :], x0_scr, sem_row.at[0]).wait()
    pltpu.make_async_copy(
        x_any.at[pl.ds(v1, 1), :], xa_scr, sem_row.at[1]).wait()
    pltpu.make_async_copy(
        x_any.at[pl.ds(v2, 1), :], xb_scr, sem_row.at[2]).wait()
    f1 = has1.astype(jnp.float32)
    f2 = has2.astype(jnp.float32)
    cnt = f1 + f2
    acc = f1 * xa_scr[...] + f2 * xb_scr[...]
    row0 = jnp.where(cnt > 0, acc / jnp.maximum(cnt, 1.0), x0_scr[...])
    w2 = w_ref[IN_F:, :]
    y0 = (
        jnp.dot(xbuf[0:1, :], w1, preferred_element_type=jnp.float32)
        + jnp.dot(row0, w2, preferred_element_type=jnp.float32)
        + bias
    )
    obuf[0:1, :] = jnp.maximum(y0, 0.0)
    oq_copy(0).start()

    oq_copy(3).wait()
    oq_copy(2).wait()
    oq_copy(1).wait()
    oq_copy(0).wait()


def kernel(x, adj, weight, bias):
    bias2 = bias.reshape(1, OUT_F)
    return pl.pallas_call(
        _body,
        out_shape=jax.ShapeDtypeStruct((N_NODES, OUT_F), jnp.float32),
        in_specs=[
            pl.BlockSpec(memory_space=pltpu.VMEM),
            pl.BlockSpec(memory_space=pltpu.VMEM),
            pl.BlockSpec(memory_space=pltpu.VMEM),
            pl.BlockSpec(memory_space=pl.ANY),
            pl.BlockSpec(memory_space=pl.ANY),
        ],
        out_specs=pl.BlockSpec(memory_space=pl.ANY),
        scratch_shapes=[
            pltpu.VMEM((N_NODES, IN_F), jnp.float32),
            pltpu.VMEM((N_NODES, OUT_F), jnp.float32),
            pltpu.VMEM((1, N_NODES), jnp.float32),
            pltpu.VMEM((1, N_NODES), jnp.float32),
            pltpu.VMEM((1, IN_F), jnp.float32),
            pltpu.VMEM((1, IN_F), jnp.float32),
            pltpu.VMEM((1, IN_F), jnp.float32),
            pltpu.SemaphoreType.DMA,
            pltpu.SemaphoreType.DMA((4,)),
            pltpu.SemaphoreType.DMA((4,)),
            pltpu.SemaphoreType.DMA((3,)),
        ],
    )(weight, bias2, jnp.asarray(_RAW), adj, x)
